# Initial kernel scaffold; baseline (speedup 1.0000x reference)
import jax
import jax.numpy as jnp
from jax.experimental import pallas as pl


def kernel(x, index, W, b, logstd):
    raise NotImplementedError("write your pallas kernel here")


# speedup vs baseline: 1.3513x; 1.3513x over previous
import functools

import jax
import jax.numpy as jnp
from jax import lax
from jax.experimental import pallas as pl
from jax.experimental.pallas import tpu as pltpu
from jax.experimental.pallas import tpu_sc as plsc

E = 8
B = 8192
D_IN = 2048
D_OUT = 2048

BM = 256
NB = B // BM + E
NPAD = NB * BM

try:
    _info = plsc.get_sparse_core_info()
    NC, NS = int(_info.num_cores), int(_info.num_subcores)
except Exception:
    NC, NS = 2, 16
NW = NC * NS
RPW = NPAD // NW
CH = 32
NCHUNK = RPW // CH

_MESH = dict(mesh=plsc.VectorSubcoreMesh(core_axis_name="c", subcore_axis_name="s"))


def _gather_body(x_hbm, src_hbm, out_hbm, idx_v, buf_v, sem):
    wid = lax.axis_index("s") * NC + lax.axis_index("c")
    base = wid * RPW
    for c in range(NCHUNK):
        off = base + c * CH
        pltpu.sync_copy(src_hbm.at[pl.ds(off, CH)], idx_v)
        pltpu.async_copy(x_hbm.at[idx_v], buf_v, sem).wait()
        pltpu.sync_copy(buf_v, out_hbm.at[pl.ds(off, CH)])


_gather = pl.kernel(
    _gather_body,
    out_type=jax.ShapeDtypeStruct((NPAD, D_IN), jnp.float32),
    scratch_types=[
        pltpu.VMEM((CH,), jnp.int32),
        pltpu.VMEM((CH, D_IN), jnp.float32),
        pltpu.SemaphoreType.DMA,
    ],
    **_MESH,
)


def _scatter_body(ys_hbm, ys2_hbm, dst_hbm, out1_hbm, out2_hbm, idx_v, buf_v, sem):
    wid = lax.axis_index("s") * NC + lax.axis_index("c")
    base = wid * RPW
    for c in range(NCHUNK):
        off = base + c * CH
        pltpu.sync_copy(dst_hbm.at[pl.ds(off, CH)], idx_v)
        pltpu.sync_copy(ys_hbm.at[pl.ds(off, CH)], buf_v)
        pltpu.async_copy(buf_v, out1_hbm.at[idx_v], sem).wait()
        pltpu.sync_copy(ys2_hbm.at[pl.ds(off, CH)], buf_v)
        pltpu.async_copy(buf_v, out2_hbm.at[idx_v], sem).wait()


_scatter = pl.kernel(
    _scatter_body,
    out_type=[
        jax.ShapeDtypeStruct((NPAD, D_OUT), jnp.float32),
        jax.ShapeDtypeStruct((NPAD, D_OUT), jnp.float32),
    ],
    scratch_types=[
        pltpu.VMEM((CH,), jnp.int32),
        pltpu.VMEM((CH, D_OUT), jnp.float32),
        pltpu.SemaphoreType.DMA,
    ],
    **_MESH,
)


def _mm_body(bexp_ref, xs_ref, w_ref, b_ref, ls_ref, ys_ref, ys2_ref):
    w = w_ref[0]
    y = lax.dot_general(
        xs_ref[...], w, (((1,), (1,)), ((), ())),
        preferred_element_type=jnp.float32,
    )
    ys_ref[...] = y + b_ref[0]
    ys2_ref[...] = jnp.broadcast_to(jnp.exp(ls_ref[0]), (BM, D_OUT))


_mm_grid = pltpu.PrefetchScalarGridSpec(
    num_scalar_prefetch=1,
    grid=(NB,),
    in_specs=[
        pl.BlockSpec((BM, D_IN), lambda j, be: (j, 0)),
        pl.BlockSpec((1, D_OUT, D_IN), lambda j, be: (be[j], 0, 0)),
        pl.BlockSpec((1, 1, D_OUT), lambda j, be: (be[j], 0, 0)),
        pl.BlockSpec((1, 1, D_OUT), lambda j, be: (be[j], 0, 0)),
    ],
    out_specs=[
        pl.BlockSpec((BM, D_OUT), lambda j, be: (j, 0)),
        pl.BlockSpec((BM, D_OUT), lambda j, be: (j, 0)),
    ],
)

_mm = pl.pallas_call(
    _mm_body,
    grid_spec=_mm_grid,
    out_shape=[
        jax.ShapeDtypeStruct((NPAD, D_OUT), jnp.float32),
        jax.ShapeDtypeStruct((NPAD, D_OUT), jnp.float32),
    ],
)


def kernel(x, index, W, b, logstd):
    idx = index.astype(jnp.int32)

    oh = (idx[:, None] == jnp.arange(E, dtype=jnp.int32)[None, :]).astype(jnp.int32)
    counts = jnp.sum(oh, axis=0)
    nblk = (counts + BM - 1) // BM
    bends = jnp.cumsum(nblk)
    astart = ((bends - nblk) * BM).astype(jnp.int32)
    rank = jnp.cumsum(oh, axis=0) - 1
    myrank = jnp.take_along_axis(rank, idx[:, None], axis=1)[:, 0]
    pos = astart[idx] + myrank
    ar = jnp.arange(B, dtype=jnp.int32)
    src = jnp.zeros((NPAD,), jnp.int32).at[pos].set(ar)
    dst = (B + (jnp.arange(NPAD, dtype=jnp.int32) % (NPAD - B))).at[pos].set(ar)
    bexp = jnp.minimum(
        jnp.searchsorted(bends, jnp.arange(NB, dtype=jnp.int32), side="right"), E - 1
    ).astype(jnp.int32)

    xs = _gather(x, src)
    ys, ys2 = _mm(bexp, xs, W, b[:, None, :], logstd[:, None, :])
    out1p, out2p = _scatter(ys, ys2, dst)
    return (out1p[:B], out2p[:B])
